# baseline (speedup 1.0000x reference)
import functools

import jax
import jax.numpy as jnp
from jax import lax
from jax.experimental import pallas as pl
from jax.experimental.pallas import tpu as pltpu
from jax.experimental.pallas import tpu_sc as plsc

_NC = 2
_NS = 16
_NW = _NC * _NS
_CH = 64


def kernel(x, embed_table):
    b, s, d = x.shape
    e = embed_table.shape[-1]
    rows = b * s
    rpw = rows // _NW
    n_ch = rpw // _CH
    xf = x.reshape(rows, d)

    @functools.partial(
        pl.kernel,
        mesh=plsc.VectorSubcoreMesh(core_axis_name="c", subcore_axis_name="s"),
        out_type=jax.ShapeDtypeStruct((rows, d + e), x.dtype),
        scratch_types=[
            pltpu.VMEM((_CH, d), jnp.float32),
            pltpu.VMEM((_CH, e), jnp.float32),
        ],
    )
    def sc_concat(x_hbm, t_hbm, out_hbm, xbuf, ebuf):
        wid = lax.axis_index("s") * _NC + lax.axis_index("c")
        base = wid * rpw
        sbase = lax.rem(base, s)

        def body(i, carry):
            r = base + i * _CH
            sr = sbase + i * _CH
            pltpu.sync_copy(x_hbm.at[pl.ds(r, _CH)], xbuf)
            pltpu.sync_copy(t_hbm.at[pl.ds(sr, _CH)], ebuf)
            pltpu.sync_copy(xbuf, out_hbm.at[pl.ds(r, _CH), pl.ds(0, d)])
            pltpu.sync_copy(ebuf, out_hbm.at[pl.ds(r, _CH), pl.ds(d, e)])
            return carry

        lax.fori_loop(0, n_ch, body, 0)

    out = sc_concat(xf, embed_table)
    return out.reshape(b, s, d + e)


# speedup vs baseline: 1.1218x; 1.1218x over previous
import functools

import jax
import jax.numpy as jnp
from jax import lax
from jax.experimental import pallas as pl
from jax.experimental.pallas import tpu as pltpu
from jax.experimental.pallas import tpu_sc as plsc

_NC = 2
_NS = 16
_NW = _NC * _NS
_CH = 32
_NBUF = 3


def kernel(x, embed_table):
    b, s, d = x.shape
    e = embed_table.shape[-1]
    rows = b * s
    rpw = rows // _NW
    n_ch = rpw // _CH
    xf = x.reshape(rows, d)

    scratch = (
        [pltpu.VMEM((_CH, d), jnp.float32) for _ in range(_NBUF)]
        + [pltpu.VMEM((_CH, e), jnp.float32) for _ in range(_NBUF)]
        + [pltpu.SemaphoreType.DMA for _ in range(2 * _NBUF)]
    )

    @functools.partial(
        pl.kernel,
        mesh=plsc.VectorSubcoreMesh(core_axis_name="c", subcore_axis_name="s"),
        out_type=jax.ShapeDtypeStruct((rows, d + e), x.dtype),
        scratch_types=scratch,
    )
    def sc_concat(x_hbm, t_hbm, out_hbm, *refs):
        xb = refs[0:_NBUF]
        eb = refs[_NBUF:2 * _NBUF]
        sin = refs[2 * _NBUF:2 * _NBUF + _NBUF]
        sout = refs[2 * _NBUF + _NBUF:]

        wid = lax.axis_index("s") * _NC + lax.axis_index("c")
        base = wid * rpw
        sbase = lax.rem(base, s)

        gathers = [None] * n_ch
        scatters = [None] * n_ch

        def start_gather(i):
            sl = i % _NBUF
            r = base + i * _CH
            sr = sbase + i * _CH
            gathers[i] = (
                pltpu.async_copy(x_hbm.at[pl.ds(r, _CH)], xb[sl], sin[sl]),
                pltpu.async_copy(t_hbm.at[pl.ds(sr, _CH)], eb[sl], sin[sl]),
            )

        def start_scatter(i):
            sl = i % _NBUF
            r = base + i * _CH
            scatters[i] = (
                pltpu.async_copy(
                    xb[sl], out_hbm.at[pl.ds(r, _CH), pl.ds(0, d)], sout[sl]
                ),
                pltpu.async_copy(
                    eb[sl], out_hbm.at[pl.ds(r, _CH), pl.ds(d, e)], sout[sl]
                ),
            )

        start_gather(0)
        if n_ch > 1:
            start_gather(1)
        for i in range(n_ch):
            for dsc in gathers[i]:
                dsc.wait()
            start_scatter(i)
            if i + 2 < n_ch:
                if i >= 1:
                    for dsc in scatters[i - 1]:
                        dsc.wait()
                start_gather(i + 2)
        for i in range(max(0, n_ch - _NBUF), n_ch):
            for dsc in scatters[i]:
                dsc.wait()

    out = sc_concat(xf, embed_table)
    return out.reshape(b, s, d + e)
